# baseline (speedup 1.0000x reference)
import functools

import jax
import jax.numpy as jnp
from jax import lax
from jax.experimental import pallas as pl
from jax.experimental.pallas import tpu as pltpu
from jax.experimental.pallas import tpu_sc as plsc

N_NODES = 10000
N_EDGES = 320000
D_FEAT = 128

_NC = 2
_NS = 16
_L = 16
_NW = _NC * _NS
_E_PER_W = N_EDGES // _NW
_B = 80
_CHUNKS = _E_PER_W // _B
_GROUPS = _B // _L
_NBUF = 3
_DW = D_FEAT // 2
_KS = _DW // _L

_HI_MASK = -65536


def _sc_body(x_hbm, src_hbm, dst_hbm, out_hbm,
             x_sp, sidx_all, didx_all,
             srows0, srows1, srows2, drows0, drows1, drows2,
             outv0, outv1, outv2,
             sem_s0, sem_s1, sem_s2, sem_d0, sem_d1, sem_d2):
    sid = lax.axis_index("s")
    wid = sid * _NC + lax.axis_index("c")
    base_w = wid * _E_PER_W

    rows_per_tile = N_NODES // _NS
    pltpu.sync_copy(x_hbm.at[pl.ds(sid * rows_per_tile, rows_per_tile)],
                    x_sp.at[pl.ds(sid * rows_per_tile, rows_per_tile)])

    pltpu.sync_copy(src_hbm.at[pl.ds(base_w, _E_PER_W)], sidx_all)
    pltpu.sync_copy(dst_hbm.at[pl.ds(base_w, _E_PER_W)], didx_all)
    plsc.subcore_barrier()

    srows = (srows0, srows1, srows2)
    drows = (drows0, drows1, drows2)
    outv = (outv0, outv1, outv2)
    sem_s = (sem_s0, sem_s1, sem_s2)
    sem_d = (sem_d0, sem_d1, sem_d2)

    def start(c, b):
        pltpu.async_copy(x_sp.at[sidx_all.at[pl.ds(c * _B, _B)]],
                         srows[b], sem_s[b])
        pltpu.async_copy(x_sp.at[didx_all.at[pl.ds(c * _B, _B)]],
                         drows[b], sem_d[b])

    def wait(b):
        pltpu.make_async_copy(x_sp.at[sidx_all.at[pl.ds(0, _B)]],
                              srows[b], sem_s[b]).wait()
        pltpu.make_async_copy(x_sp.at[didx_all.at[pl.ds(0, _B)]],
                              drows[b], sem_d[b]).wait()


    def compute(b):
        sr, dr, ov = srows[b], drows[b], outv[b]

        def group_body(g):
            lane = lax.iota(jnp.int32, _L)
            tot = jnp.zeros((_L,), jnp.float32)
            for j in range(_L):
                e = g * _L + j
                acc = jnp.zeros((_L,), jnp.float32)
                for k in range(_KS):
                    sv = sr[e, pl.ds(k * 2 * _L, 2 * _L)]
                    dv = dr[e, pl.ds(k * 2 * _L, 2 * _L)]
                    p = plsc.bitcast(sv * dv, jnp.int32)
                    acc = acc + plsc.bitcast(p << 16, jnp.float32)
                    acc = acc + plsc.bitcast(p, jnp.float32)
                tot = jnp.where(lane == j, jnp.sum(acc), tot)
            ov[pl.ds(g * _L, _L)] = tot

        plsc.parallel_loop(0, _GROUPS, 1, unroll=1)(group_body)

    def store(c, b):
        pltpu.sync_copy(outv[b], out_hbm.at[pl.ds(base_w + c * _B, _B)])

    for b in range(_NBUF):
        start(b, b)

    _ROUNDS = _CHUNKS // _NBUF

    def round_body(p, carry):
        for b in range(_NBUF):
            c = _NBUF * p + b
            wait(b)
            compute(b)
            store(c, b)

            @pl.when(c + _NBUF < _CHUNKS)
            def _():
                start(c + _NBUF, b)

        return carry

    lax.fori_loop(0, _ROUNDS, round_body, 0, unroll=False)
    for c in range(_NBUF * _ROUNDS, _CHUNKS):
        b = c % _NBUF
        wait(b)
        compute(b)
        store(c, b)


@jax.jit
def _score(x_bf, src, dst):
    mesh = plsc.VectorSubcoreMesh(core_axis_name="c", subcore_axis_name="s")
    f = functools.partial(
        pl.kernel,
        mesh=mesh,
        compiler_params=pltpu.CompilerParams(
            needs_layout_passes=False, use_tc_tiling_on_sc=False),
        out_type=jax.ShapeDtypeStruct((N_EDGES,), jnp.float32),
        scratch_types=[
            pltpu.VMEM_SHARED((N_NODES, D_FEAT), jnp.bfloat16),
            pltpu.VMEM((_E_PER_W,), jnp.int32),
            pltpu.VMEM((_E_PER_W,), jnp.int32),
            pltpu.VMEM((_B, D_FEAT), jnp.bfloat16),
            pltpu.VMEM((_B, D_FEAT), jnp.bfloat16),
            pltpu.VMEM((_B, D_FEAT), jnp.bfloat16),
            pltpu.VMEM((_B, D_FEAT), jnp.bfloat16),
            pltpu.VMEM((_B, D_FEAT), jnp.bfloat16),
            pltpu.VMEM((_B, D_FEAT), jnp.bfloat16),
            pltpu.VMEM((_B,), jnp.float32),
            pltpu.VMEM((_B,), jnp.float32),
            pltpu.VMEM((_B,), jnp.float32),
            pltpu.SemaphoreType.DMA,
            pltpu.SemaphoreType.DMA,
            pltpu.SemaphoreType.DMA,
            pltpu.SemaphoreType.DMA,
            pltpu.SemaphoreType.DMA,
            pltpu.SemaphoreType.DMA,
        ],
    )(_sc_body)
    return f(x_bf, src, dst)


def kernel(x, edge_index):
    src = edge_index[0].astype(jnp.int32)
    dst = edge_index[1].astype(jnp.int32)
    x_bf = x.astype(jnp.bfloat16)
    score = _score(x_bf, src, dst)
    return score.reshape(N_EDGES, 1)


# speedup vs baseline: 2.1299x; 2.1299x over previous
import functools

import jax
import jax.numpy as jnp
from jax import lax
from jax.experimental import pallas as pl
from jax.experimental.pallas import tpu as pltpu
from jax.experimental.pallas import tpu_sc as plsc

N_NODES = 10000
N_EDGES = 320000
D_FEAT = 128

_NC = 2
_NS = 16
_L = 16
_NW = _NC * _NS
_E_PER_W = N_EDGES // _NW
_B = 80
_CHUNKS = _E_PER_W // _B
_GROUPS = _B // _L
_KS = D_FEAT // (2 * _L)


def _sc_body(x_hbm, src_hbm, dst_hbm, out_hbm,
             x_sp, sidx_all, didx_all,
             srows0, srows1, drows0, drows1, outv0, outv1,
             sem_s0, sem_s1, sem_d0, sem_d1):
    sid = lax.axis_index("s")
    wid = sid * _NC + lax.axis_index("c")
    base_w = wid * _E_PER_W

    rows_per_tile = N_NODES // _NS
    pltpu.sync_copy(x_hbm.at[pl.ds(sid * rows_per_tile, rows_per_tile)],
                    x_sp.at[pl.ds(sid * rows_per_tile, rows_per_tile)])

    pltpu.sync_copy(src_hbm.at[pl.ds(base_w, _E_PER_W)], sidx_all)
    pltpu.sync_copy(dst_hbm.at[pl.ds(base_w, _E_PER_W)], didx_all)
    plsc.subcore_barrier()

    srows = (srows0, srows1)
    drows = (drows0, drows1)
    outv = (outv0, outv1)
    sem_s = (sem_s0, sem_s1)
    sem_d = (sem_d0, sem_d1)

    def start(c, b):
        pltpu.async_copy(x_sp.at[sidx_all.at[pl.ds(c * _B, _B)]],
                         srows[b], sem_s[b])
        pltpu.async_copy(x_sp.at[didx_all.at[pl.ds(c * _B, _B)]],
                         drows[b], sem_d[b])

    def wait(b):
        pltpu.make_async_copy(x_sp.at[sidx_all.at[pl.ds(0, _B)]],
                              srows[b], sem_s[b]).wait()
        pltpu.make_async_copy(x_sp.at[didx_all.at[pl.ds(0, _B)]],
                              drows[b], sem_d[b]).wait()


    def compute(b):
        sr, dr, ov = srows[b], drows[b], outv[b]

        def group_body(g, carry):
            lane = lax.iota(jnp.int32, _L)
            tot = jnp.zeros((_L,), jnp.float32)
            for j in range(_L):
                e = g * _L + j
                acc = jnp.zeros((_L,), jnp.float32)
                for k in range(_KS):
                    sv = sr[e, pl.ds(k * 2 * _L, 2 * _L)]
                    dv = dr[e, pl.ds(k * 2 * _L, 2 * _L)]
                    p = plsc.bitcast(sv * dv, jnp.int32)
                    acc = acc + plsc.bitcast(p << 16, jnp.float32)
                    acc = acc + plsc.bitcast(p, jnp.float32)
                tot = jnp.where(lane == j, jnp.sum(acc), tot)
            ov[pl.ds(g * _L, _L)] = tot
            return carry

        lax.fori_loop(0, _GROUPS, group_body, 0, unroll=False)

    def store(c, b):
        pltpu.sync_copy(outv[b], out_hbm.at[pl.ds(base_w + c * _B, _B)])

    start(0, 0)
    start(1, 1)

    def pair_body(p, carry):
        for b in (0, 1):
            c = 2 * p + b
            wait(b)
            compute(b)
            store(c, b)

            @pl.when(c + 2 < _CHUNKS)
            def _():
                start(c + 2, b)

        return carry

    lax.fori_loop(0, (_CHUNKS - 1) // 2, pair_body, 0, unroll=False)
    wait(0)
    compute(0)
    store(_CHUNKS - 1, 0)


@jax.jit
def _score(x_bf, src, dst):
    mesh = plsc.VectorSubcoreMesh(core_axis_name="c", subcore_axis_name="s")
    f = functools.partial(
        pl.kernel,
        mesh=mesh,
        compiler_params=pltpu.CompilerParams(
            needs_layout_passes=False, use_tc_tiling_on_sc=False),
        out_type=jax.ShapeDtypeStruct((N_EDGES,), jnp.float32),
        scratch_types=[
            pltpu.VMEM_SHARED((N_NODES, D_FEAT), jnp.bfloat16),
            pltpu.VMEM((_E_PER_W,), jnp.int32),
            pltpu.VMEM((_E_PER_W,), jnp.int32),
            pltpu.VMEM((_B, D_FEAT), jnp.bfloat16),
            pltpu.VMEM((_B, D_FEAT), jnp.bfloat16),
            pltpu.VMEM((_B, D_FEAT), jnp.bfloat16),
            pltpu.VMEM((_B, D_FEAT), jnp.bfloat16),
            pltpu.VMEM((_B,), jnp.float32),
            pltpu.VMEM((_B,), jnp.float32),
            pltpu.SemaphoreType.DMA,
            pltpu.SemaphoreType.DMA,
            pltpu.SemaphoreType.DMA,
            pltpu.SemaphoreType.DMA,
        ],
    )(_sc_body)
    return f(x_bf, src, dst)


def kernel(x, edge_index):
    src = edge_index[0].astype(jnp.int32)
    dst = edge_index[1].astype(jnp.int32)
    x_bf = x.astype(jnp.bfloat16)
    score = _score(x_bf, src, dst)
    return score.reshape(N_EDGES, 1)
